# Initial kernel scaffold; baseline (speedup 1.0000x reference)
import jax
import jax.numpy as jnp
from jax.experimental import pallas as pl


def kernel(pts, timestamps, grid_0_0, grid_0_1, grid_0_2, grid_0_3, grid_0_4, grid_0_5, grid_1_0, grid_1_1, grid_1_2, grid_1_3, grid_1_4, grid_1_5, grid_2_0, grid_2_1, grid_2_2, grid_2_3, grid_2_4, grid_2_5):
    raise NotImplementedError("write your pallas kernel here")


# speedup vs baseline: 18.3513x; 18.3513x over previous
import functools
import itertools

import numpy as np
import jax
import jax.numpy as jnp
from jax import lax
from jax.experimental import pallas as pl
from jax.experimental.pallas import tpu as pltpu
from jax.experimental.pallas import tpu_sc as plsc

_BOUNDS = 1.6
_DURATION = 32.0
_C = 64
_BASE_RES = [64, 64, 64, 32]
_MULTIRES = [1, 2, 4]
_COMBS = list(itertools.combinations(range(4), 2))
_N = 200000

_NC, _NS = 2, 16
_NW = _NC * _NS
_P = 64
_CHUNKS = (_N + _NW * _P - 1) // (_NW * _P)
_NPAD = _NW * _P * _CHUNKS

_KSCALE = float(np.float32(2.0) / (np.float32(-_BOUNDS) - np.float32(_BOUNDS)))
_C16 = float(np.float32(_BOUNDS))

_PLANES = []
_base = 0
for _s, _r in enumerate(_MULTIRES):
    _reso = [_BASE_RES[0] * _r, _BASE_RES[1] * _r, _BASE_RES[2] * _r, _BASE_RES[3]]
    for _ci, (_c0, _c1) in enumerate(_COMBS):
        _W, _H = _reso[_c0], _reso[_c1]
        _PLANES.append((_base, _W, _H, _c0, _c1, 64 * _s, _ci == 0))
        _base += (_H + 1) * _W
_VTOT = _base


def _build_table(grids):
    tabs = []
    for g in grids:
        g = jnp.transpose(g[0], (1, 2, 0))
        gx = jnp.concatenate([g, g[:, -1:, :]], axis=1)
        dup = jnp.concatenate([gx[:, :-1, :], gx[:, 1:, :]], axis=2)
        dup = jnp.concatenate([dup, dup[-1:]], axis=0)
        tabs.append(dup.reshape(-1, 2 * _C))
    return jnp.concatenate(tabs, axis=0)


@functools.lru_cache(maxsize=1)
def _get_sc_kernel():
    mesh = plsc.VectorSubcoreMesh(core_axis_name="c", subcore_axis_name="s",
                                  num_cores=_NC, num_subcores=_NS)
    return functools.partial(
        pl.kernel,
        out_type=jax.ShapeDtypeStruct((_NPAD * 192,), jnp.float32),
        mesh=mesh,
        scratch_types=[
            pltpu.VMEM((4 * _P,), jnp.float32),
            pltpu.VMEM((4, _P), jnp.float32),
            pltpu.VMEM((_P,), jnp.int32),
            pltpu.VMEM((_P,), jnp.int32),
            pltpu.VMEM((4, _P, 16), jnp.float32),
            pltpu.VMEM((_P, 128), jnp.float32),
            pltpu.VMEM((_P, 128), jnp.float32),
            pltpu.VMEM((_P * 192,), jnp.float32),
            pltpu.SemaphoreType.DMA,
        ],
    )(_hexplane_sc_body)


def _hexplane_sc_body(coords_hbm, table_hbm, out_hbm,
                      crd_v, p4_v, ia_v, ib_v, w_v, ra_v, rb_v, acc_v, sem):
    wid = lax.axis_index("s") * _NC + lax.axis_index("c")

    def chunk_body(t, _):
        blk = wid * _CHUNKS + t
        base = blk * _P
        pltpu.sync_copy(coords_hbm.at[pl.ds(blk * 4 * _P, 4 * _P)], crd_v)

        def norm_body(g, _):
            sl = pl.ds(g * 16, 16)
            for c in range(3):
                p4_v[c, sl] = (crd_v[pl.ds(c * _P + g * 16, 16)] - _C16) * _KSCALE - 1.0
            p4_v[3, sl] = (2.0 * crd_v[pl.ds(3 * _P + g * 16, 16)]
                           * _DURATION / (_DURATION - 1.0) - 1.0)
            return ()

        lax.fori_loop(0, _P // 16, norm_body, ())

        for (pbase, W, H, c0, c1, out_off, first) in _PLANES:
            wm1 = float(W - 1)
            hm1 = float(H - 1)

            def idx_body(g, _, pbase=pbase, W=W, c0=c0, c1=c1, wm1=wm1, hm1=hm1):
                sl = pl.ds(g * 16, 16)
                x = (p4_v[c0, sl] + 1.0) * 0.5 * wm1
                y = (p4_v[c1, sl] + 1.0) * 0.5 * hm1
                x = jnp.clip(x, 0.0, wm1)
                y = jnp.clip(y, 0.0, hm1)
                x0i = x.astype(jnp.int32)
                y0i = y.astype(jnp.int32)
                wx = x - x0i.astype(jnp.float32)
                wy = y - y0i.astype(jnp.float32)
                ia = pbase + y0i * W + x0i
                ia_v[sl] = ia
                ib_v[sl] = ia + W
                u = 1.0 - wx
                v = 1.0 - wy
                w00 = u * v
                w01 = wx * v
                w10 = u * wy
                w11 = wx * wy
                for k in range(16):
                    i = g * 16 + k
                    w_v[0, i, :] = jnp.full((16,), w00[k], jnp.float32)
                    w_v[1, i, :] = jnp.full((16,), w01[k], jnp.float32)
                    w_v[2, i, :] = jnp.full((16,), w10[k], jnp.float32)
                    w_v[3, i, :] = jnp.full((16,), w11[k], jnp.float32)
                return ()

            lax.fori_loop(0, _P // 16, idx_body, ())

            cpa = pltpu.async_copy(table_hbm.at[ia_v], ra_v, sem)
            cpb = pltpu.async_copy(table_hbm.at[ib_v], rb_v, sem)
            cpa.wait()
            cpb.wait()

            def acc_body(i, _, out_off=out_off, first=first):
                a = w_v[0, i, :]
                b = w_v[1, i, :]
                c = w_v[2, i, :]
                d = w_v[3, i, :]
                for j in range(4):
                    s0 = pl.ds(j * 16, 16)
                    s1 = pl.ds(64 + j * 16, 16)
                    val = (a * ra_v[i, s0] + b * ra_v[i, s1]
                           + c * rb_v[i, s0] + d * rb_v[i, s1])
                    tsl = pl.ds(i * 192 + out_off + j * 16, 16)
                    if first:
                        acc_v[tsl] = val
                    else:
                        acc_v[tsl] = acc_v[tsl] + val
                return ()

            lax.fori_loop(0, _P, acc_body, ())

        pltpu.sync_copy(acc_v, out_hbm.at[pl.ds(base * 192, _P * 192)])
        return ()

    lax.fori_loop(0, _CHUNKS, chunk_body, ())


def kernel(pts, timestamps,
           grid_0_0, grid_0_1, grid_0_2, grid_0_3, grid_0_4, grid_0_5,
           grid_1_0, grid_1_1, grid_1_2, grid_1_3, grid_1_4, grid_1_5,
           grid_2_0, grid_2_1, grid_2_2, grid_2_3, grid_2_4, grid_2_5):
    grids = [grid_0_0, grid_0_1, grid_0_2, grid_0_3, grid_0_4, grid_0_5,
             grid_1_0, grid_1_1, grid_1_2, grid_1_3, grid_1_4, grid_1_5,
             grid_2_0, grid_2_1, grid_2_2, grid_2_3, grid_2_4, grid_2_5]
    table = _build_table(grids)
    coords = jnp.concatenate([pts, timestamps], axis=1).T
    coords = jnp.pad(coords, ((0, 0), (0, _NPAD - _N)))
    coords = jnp.transpose(coords.reshape(4, _NPAD // _P, _P), (1, 0, 2)).reshape(-1)
    out = _get_sc_kernel()(coords, table)
    return out.reshape(_NPAD, 192)[:_N]


# speedup vs baseline: 24.8870x; 1.3561x over previous
import functools
import itertools

import numpy as np
import jax
import jax.numpy as jnp
from jax import lax
from jax.experimental import pallas as pl
from jax.experimental.pallas import tpu as pltpu
from jax.experimental.pallas import tpu_sc as plsc

_BOUNDS = 1.6
_DURATION = 32.0
_C = 64
_BASE_RES = [64, 64, 64, 32]
_MULTIRES = [1, 2, 4]
_COMBS = list(itertools.combinations(range(4), 2))
_N = 200000

_NC, _NS = 2, 16
_NW = _NC * _NS
_P = 64
_CHUNKS = (_N + _NW * _P - 1) // (_NW * _P)
_NPAD = _NW * _P * _CHUNKS

_KSCALE = float(np.float32(2.0) / (np.float32(-_BOUNDS) - np.float32(_BOUNDS)))
_C16 = float(np.float32(_BOUNDS))

_PLANES = []
_base = 0
for _s, _r in enumerate(_MULTIRES):
    _reso = [_BASE_RES[0] * _r, _BASE_RES[1] * _r, _BASE_RES[2] * _r, _BASE_RES[3]]
    for _ci, (_c0, _c1) in enumerate(_COMBS):
        _W, _H = _reso[_c0], _reso[_c1]
        _PLANES.append((_base, _W, _H, _c0, _c1, 64 * _s, _ci == 0))
        _base += (_H + 1) * _W
_VTOT = _base


def _build_table(grids):
    tabs = []
    for g in grids:
        g = jnp.transpose(g[0], (1, 2, 0))
        gx = jnp.concatenate([g, g[:, -1:, :]], axis=1)
        dup = jnp.concatenate([gx[:, :-1, :], gx[:, 1:, :]], axis=2)
        dup = jnp.concatenate([dup, dup[-1:]], axis=0)
        tabs.append(dup.reshape(-1, 2 * _C))
    return jnp.concatenate(tabs, axis=0)


@functools.lru_cache(maxsize=1)
def _get_sc_kernel():
    mesh = plsc.VectorSubcoreMesh(core_axis_name="c", subcore_axis_name="s",
                                  num_cores=_NC, num_subcores=_NS)
    return functools.partial(
        pl.kernel,
        out_type=jax.ShapeDtypeStruct((_NPAD * 192,), jnp.float32),
        mesh=mesh,
        scratch_types=[
            pltpu.VMEM((4 * _P,), jnp.float32),
            pltpu.VMEM((4, _P), jnp.float32),
            pltpu.VMEM((2, _P), jnp.int32),
            pltpu.VMEM((2, _P), jnp.int32),
            pltpu.VMEM((2, 4, _P, 16), jnp.float32),
            pltpu.VMEM((2, _P, 128), jnp.float32),
            pltpu.VMEM((2, _P, 128), jnp.float32),
            pltpu.VMEM((_P * 192,), jnp.float32),
            pltpu.SemaphoreType.DMA,
            pltpu.SemaphoreType.DMA,
        ],
    )(_hexplane_sc_body)


def _hexplane_sc_body(coords_hbm, table_hbm, out_hbm,
                      crd_v, p4_v, ia_v, ib_v, w_v, ra_v, rb_v, acc_v,
                      sem0, sem1):
    wid = lax.axis_index("s") * _NC + lax.axis_index("c")
    sems = (sem0, sem1)

    def chunk_body(t, _):
        blk = wid * _CHUNKS + t
        base = blk * _P
        pltpu.sync_copy(coords_hbm.at[pl.ds(blk * 4 * _P, 4 * _P)], crd_v)

        def norm_body(g, _):
            sl = pl.ds(g * 16, 16)
            for c in range(3):
                p4_v[c, sl] = (crd_v[pl.ds(c * _P + g * 16, 16)] - _C16) * _KSCALE - 1.0
            p4_v[3, sl] = (2.0 * crd_v[pl.ds(3 * _P + g * 16, 16)]
                           * _DURATION / (_DURATION - 1.0) - 1.0)
            return ()

        lax.fori_loop(0, _P // 16, norm_body, ())

        def stage_a(k, pb):
            pbase, W, H, c0, c1, _, _ = _PLANES[k]
            wm1 = float(W - 1)
            hm1 = float(H - 1)

            def idx_body(g, _, pbase=pbase, W=W, c0=c0, c1=c1,
                         wm1=wm1, hm1=hm1, pb=pb):
                sl = pl.ds(g * 16, 16)
                x = (p4_v[c0, sl] + 1.0) * 0.5 * wm1
                y = (p4_v[c1, sl] + 1.0) * 0.5 * hm1
                x = jnp.clip(x, 0.0, wm1)
                y = jnp.clip(y, 0.0, hm1)
                x0i = x.astype(jnp.int32)
                y0i = y.astype(jnp.int32)
                wx = x - x0i.astype(jnp.float32)
                wy = y - y0i.astype(jnp.float32)
                ia = pbase + y0i * W + x0i
                ia_v[pb, sl] = ia
                ib_v[pb, sl] = ia + W
                u = 1.0 - wx
                v = 1.0 - wy
                w00 = u * v
                w01 = wx * v
                w10 = u * wy
                w11 = wx * wy
                for k16 in range(16):
                    i = g * 16 + k16
                    w_v[pb, 0, i, :] = jnp.full((16,), w00[k16], jnp.float32)
                    w_v[pb, 1, i, :] = jnp.full((16,), w01[k16], jnp.float32)
                    w_v[pb, 2, i, :] = jnp.full((16,), w10[k16], jnp.float32)
                    w_v[pb, 3, i, :] = jnp.full((16,), w11[k16], jnp.float32)
                return ()

            lax.fori_loop(0, _P // 16, idx_body, ())

        def fire(k, pb):
            cpa = pltpu.async_copy(table_hbm.at[ia_v.at[pb]], ra_v.at[pb], sems[pb])
            cpb = pltpu.async_copy(table_hbm.at[ib_v.at[pb]], rb_v.at[pb], sems[pb])
            return cpa, cpb

        def accumulate(k, pb):
            _, _, _, _, _, out_off, first = _PLANES[k]

            def acc_body(i, _, out_off=out_off, first=first, pb=pb):
                a = w_v[pb, 0, i, :]
                b = w_v[pb, 1, i, :]
                c = w_v[pb, 2, i, :]
                d = w_v[pb, 3, i, :]
                for j in range(4):
                    s0 = pl.ds(j * 16, 16)
                    s1 = pl.ds(64 + j * 16, 16)
                    val = (a * ra_v[pb, i, s0] + b * ra_v[pb, i, s1]
                           + c * rb_v[pb, i, s0] + d * rb_v[pb, i, s1])
                    tsl = pl.ds(i * 192 + out_off + j * 16, 16)
                    if first:
                        acc_v[tsl] = val
                    else:
                        acc_v[tsl] = acc_v[tsl] + val
                return ()

            lax.fori_loop(0, _P, acc_body, ())

        stage_a(0, 0)
        descs = {0: fire(0, 0)}
        for k in range(len(_PLANES)):
            cb = k % 2
            nb = 1 - cb
            if k + 1 < len(_PLANES):
                stage_a(k + 1, nb)
                descs[k + 1] = fire(k + 1, nb)
            da, db = descs.pop(k)
            da.wait()
            db.wait()
            accumulate(k, cb)

        pltpu.sync_copy(acc_v, out_hbm.at[pl.ds(base * 192, _P * 192)])
        return ()

    lax.fori_loop(0, _CHUNKS, chunk_body, ())


def kernel(pts, timestamps,
           grid_0_0, grid_0_1, grid_0_2, grid_0_3, grid_0_4, grid_0_5,
           grid_1_0, grid_1_1, grid_1_2, grid_1_3, grid_1_4, grid_1_5,
           grid_2_0, grid_2_1, grid_2_2, grid_2_3, grid_2_4, grid_2_5):
    grids = [grid_0_0, grid_0_1, grid_0_2, grid_0_3, grid_0_4, grid_0_5,
             grid_1_0, grid_1_1, grid_1_2, grid_1_3, grid_1_4, grid_1_5,
             grid_2_0, grid_2_1, grid_2_2, grid_2_3, grid_2_4, grid_2_5]
    table = _build_table(grids)
    coords = jnp.concatenate([pts, timestamps], axis=1).T
    coords = jnp.pad(coords, ((0, 0), (0, _NPAD - _N)))
    coords = jnp.transpose(coords.reshape(4, _NPAD // _P, _P), (1, 0, 2)).reshape(-1)
    out = _get_sc_kernel()(coords, table)
    return out.reshape(_NPAD, 192)[:_N]
